# Initial kernel scaffold; baseline (speedup 1.0000x reference)
import jax
import jax.numpy as jnp
from jax.experimental import pallas as pl


def kernel(x_expert_input, gate_input, gate_W, gate_b, W1, b1, W2, b2):
    raise NotImplementedError("write your pallas kernel here")


# speedup vs baseline: 1.6431x; 1.6431x over previous
import functools

import jax
import jax.numpy as jnp
from jax.experimental import pallas as pl
from jax.experimental.pallas import tpu as pltpu

N = 4096
D_GATE = 2304
D_EXP = 2304
H = 1024
O = 512
E = 8
K = 2

BN = 512


def _gate_body(gx_ref, gw_ref, gb_ref, w_ref):
    logits = jnp.dot(gx_ref[...], gw_ref[...],
                     preferred_element_type=jnp.float32) + gb_ref[...]
    iota = jax.lax.broadcasted_iota(jnp.int32, logits.shape, 1)
    m1 = jnp.max(logits, axis=-1, keepdims=True)
    i1 = jnp.min(jnp.where(logits == m1, iota, E), axis=-1, keepdims=True)
    l2 = jnp.where(iota == i1, -jnp.inf, logits)
    m2 = jnp.max(l2, axis=-1, keepdims=True)
    i2 = jnp.min(jnp.where(l2 == m2, iota, E), axis=-1, keepdims=True)
    e2 = jnp.exp(m2 - m1)
    denom = 1.0 + e2
    w1 = 1.0 / denom
    w2 = e2 / denom
    w_ref[...] = (jnp.where(iota == i1, w1, 0.0)
                  + jnp.where(iota == i2, w2, 0.0))


def _expert_body(wd_ref, x_ref, w1_ref, b1_ref, w2_ref, b2_ref, y_ref):
    e = pl.program_id(1)

    @pl.when(e == 0)
    def _():
        y_ref[...] = jnp.zeros_like(y_ref)

    h = jnp.maximum(
        jnp.dot(x_ref[...], w1_ref[0], preferred_element_type=jnp.float32)
        + b1_ref[0], 0.0)
    out = jnp.dot(h, w2_ref[0], preferred_element_type=jnp.float32) + b2_ref[0]
    wd = wd_ref[...]
    lane = jax.lax.broadcasted_iota(jnp.int32, wd.shape, 1)
    w_col = jnp.sum(jnp.where(lane == e, wd, 0.0), axis=1, keepdims=True)
    y_ref[...] += w_col * out


def kernel(x_expert_input, gate_input, gate_W, gate_b, W1, b1, W2, b2):
    w_dense = pl.pallas_call(
        _gate_body,
        grid=(N // BN,),
        in_specs=[
            pl.BlockSpec((BN, D_GATE), lambda i: (i, 0)),
            pl.BlockSpec((D_GATE, E), lambda i: (0, 0)),
            pl.BlockSpec((1, E), lambda i: (0, 0)),
        ],
        out_specs=pl.BlockSpec((BN, E), lambda i: (i, 0)),
        out_shape=jax.ShapeDtypeStruct((N, E), jnp.float32),
    )(gate_input, gate_W, gate_b.reshape(1, E))

    y = pl.pallas_call(
        _expert_body,
        grid=(N // BN, E),
        in_specs=[
            pl.BlockSpec((BN, E), lambda i, e: (i, 0)),
            pl.BlockSpec((BN, D_EXP), lambda i, e: (i, 0)),
            pl.BlockSpec((1, D_EXP, H), lambda i, e: (e, 0, 0)),
            pl.BlockSpec((1, 1, H), lambda i, e: (e, 0, 0)),
            pl.BlockSpec((1, H, O), lambda i, e: (e, 0, 0)),
            pl.BlockSpec((1, 1, O), lambda i, e: (e, 0, 0)),
        ],
        out_specs=pl.BlockSpec((BN, O), lambda i, e: (i, 0)),
        out_shape=jax.ShapeDtypeStruct((N, O), jnp.float32),
        compiler_params=pltpu.CompilerParams(
            dimension_semantics=("parallel", "arbitrary")),
    )(w_dense, x_expert_input, W1, b1.reshape(E, 1, H), W2, b2.reshape(E, 1, O))
    return y
